# baseline (speedup 1.0000x reference)
import functools

import jax
import jax.numpy as jnp
from jax import lax
from jax.experimental import pallas as pl
from jax.experimental.pallas import tpu as pltpu
from jax.experimental.pallas import tpu_sc as plsc

NC = 2
NS = 16
LANES = 16
NW = NC * NS


def _proj_body(x_ref, w_ref, os_ref, od_ref, on_ref):
    d = x_ref.shape[1]
    r = jnp.dot(x_ref[...], w_ref[...], preferred_element_type=jnp.float32)
    os_ref[...] = r[:, :d]
    od_ref[...] = r[:, d:2 * d]
    on_ref[...] = r[:, 2 * d:]


def _proj(x, wcat, block_n=2000):
    n, d = x.shape
    k = wcat.shape[1]
    return pl.pallas_call(
        _proj_body,
        grid=(n // block_n,),
        in_specs=[
            pl.BlockSpec((block_n, d), lambda i: (i, 0)),
            pl.BlockSpec((d, k), lambda i: (0, 0)),
        ],
        out_specs=[
            pl.BlockSpec((block_n, d), lambda i: (i, 0)),
            pl.BlockSpec((block_n, d), lambda i: (i, 0)),
            pl.BlockSpec((block_n, d), lambda i: (i, 0)),
        ],
        out_shape=[
            jax.ShapeDtypeStruct((n, d), jnp.float32),
            jax.ShapeDtypeStruct((n, d), jnp.float32),
            jax.ShapeDtypeStruct((n, d), jnp.float32),
        ],
    )(x, wcat)


def _gather_sum(ps, pd, src, dst, chunk=200):
    e = src.shape[0]
    d = ps.shape[1]
    per_w = e // NW
    iters = per_w // chunk
    mesh = plsc.VectorSubcoreMesh(core_axis_name="c", subcore_axis_name="s")

    @functools.partial(
        pl.kernel, mesh=mesh,
        out_type=jax.ShapeDtypeStruct((e, d), jnp.float32),
        scratch_types=[
            pltpu.VMEM((per_w,), jnp.int32),
            pltpu.VMEM((per_w,), jnp.int32),
            [pltpu.VMEM((chunk, d), jnp.float32)] * 2,
            [pltpu.VMEM((chunk, d), jnp.float32)] * 2,
            [pltpu.SemaphoreType.DMA] * 2,
            [pltpu.SemaphoreType.DMA] * 2,
            [pltpu.SemaphoreType.DMA] * 2,
        ],
    )
    def k(ps_hbm, pd_hbm, src_hbm, dst_hbm, out_hbm,
          si_v, di_v, rs_v, rd_v, sem_s, sem_d, sem_o):
        wid = lax.axis_index("s") * NC + lax.axis_index("c")
        w_base = wid * per_w

        pltpu.sync_copy(src_hbm.at[pl.ds(w_base, per_w)], si_v)
        pltpu.sync_copy(dst_hbm.at[pl.ds(w_base, per_w)], di_v)

        def fire(i, b):
            off = i * chunk
            pltpu.async_copy(ps_hbm.at[si_v.at[pl.ds(off, chunk)]],
                             rs_v[b], sem_s[b])
            pltpu.async_copy(pd_hbm.at[di_v.at[pl.ds(off, chunk)]],
                             rd_v[b], sem_d[b])

        fire(0, 0)

        @pl.loop(0, iters, step=2)
        def outer(i0):
            for b in range(2):
                i = i0 + b

                @pl.when(i < iters)
                def _():
                    @pl.when(i + 1 < iters)
                    def _():
                        @pl.when(i >= 1)
                        def _():
                            pltpu.make_async_copy(
                                rs_v[1 - b],
                                out_hbm.at[pl.ds(w_base + (i - 1) * chunk,
                                                 chunk)],
                                sem_o[1 - b]).wait()

                        fire(i + 1, 1 - b)

                    off = i * chunk
                    pltpu.make_async_copy(
                        ps_hbm.at[si_v.at[pl.ds(off, chunk)]], rs_v[b],
                        sem_s[b]).wait()
                    pltpu.make_async_copy(
                        pd_hbm.at[di_v.at[pl.ds(off, chunk)]], rd_v[b],
                        sem_d[b]).wait()

                    @pl.loop(0, chunk, unroll=4)
                    def addrow(r):
                        for j in range(d // LANES):
                            sl = pl.ds(j * LANES, LANES)
                            rs_v[b][r, sl] = rs_v[b][r, sl] + rd_v[b][r, sl]

                    pltpu.async_copy(rs_v[b],
                                     out_hbm.at[pl.ds(w_base + i * chunk,
                                                      chunk)],
                                     sem_o[b])

        for b in range(2):
            i_last = iters - 1 - (1 - b if iters % 2 == 0 else b)
            pltpu.make_async_copy(
                rs_v[b], out_hbm.at[pl.ds(w_base + i_last * chunk, chunk)],
                sem_o[b]).wait()

    return k(ps, pd, src, dst)


def _edge_mlp(gsum, ea, w1a, b1e, w2e, b2e, block_e=8000):
    e, d = gsum.shape
    de = ea.shape[1]

    def body(g_ref, ea_ref, w1a_ref, b1_ref, w2_ref, b2_ref, o_ref, op_ref):
        h = (g_ref[...]
             + jnp.dot(ea_ref[...], w1a_ref[...],
                       preferred_element_type=jnp.float32)
             + b1_ref[...])
        h = jnp.maximum(h, 0.0)
        out = jnp.dot(h, w2_ref[...],
                      preferred_element_type=jnp.float32) + b2_ref[...]
        o_ref[...] = out
        out3 = out.reshape(block_e // 8, 8, de)
        op_ref[...] = jnp.concatenate([out3[:, a, :] for a in range(8)],
                                      axis=-1)

    return pl.pallas_call(
        body,
        grid=(e // block_e,),
        in_specs=[
            pl.BlockSpec((block_e, d), lambda i: (i, 0)),
            pl.BlockSpec((block_e, de), lambda i: (i, 0)),
            pl.BlockSpec((de, d), lambda i: (0, 0)),
            pl.BlockSpec((1, d), lambda i: (0, 0)),
            pl.BlockSpec((d, de), lambda i: (0, 0)),
            pl.BlockSpec((1, de), lambda i: (0, 0)),
        ],
        out_specs=[
            pl.BlockSpec((block_e, de), lambda i: (i, 0)),
            pl.BlockSpec((block_e // 8, 8 * de), lambda i: (i, 0)),
        ],
        out_shape=[
            jax.ShapeDtypeStruct((e, de), jnp.float32),
            jax.ShapeDtypeStruct((e // 8, 8 * de), jnp.float32),
        ],
    )(gsum, ea, w1a, b1e, w2e, b2e)


def _scatter_partials(new_ea, dst, n_nodes, chunk=400):
    e, de = new_ea.shape
    per_w = e // NW
    iters = per_w // chunk
    n_pad = ((n_nodes + 8 * NS - 1) // (8 * NS)) * (8 * NS)
    rows_per_tile = n_pad // NS
    mesh = plsc.VectorSubcoreMesh(core_axis_name="c", subcore_axis_name="s")

    @functools.partial(
        pl.kernel, mesh=mesh,
        out_type=[jax.ShapeDtypeStruct((NC * n_pad, de), jnp.float32),
                  jax.ShapeDtypeStruct((NC * n_pad, de), jnp.float32)],
        scratch_types=[
            pltpu.VMEM((chunk,), jnp.int32),
            pltpu.VMEM((chunk, de), jnp.float32),
            pltpu.VMEM((chunk, de), jnp.float32),
            pltpu.VMEM((rows_per_tile, de), jnp.float32),
            pltpu.VMEM_SHARED((n_pad, de), jnp.float32),
            pltpu.VMEM_SHARED((n_pad, de), jnp.float32),
        ],
        compiler_params=pltpu.CompilerParams(use_tc_tiling_on_sc=False),
    )
    def k(ea_hbm, dst_hbm, agg_hbm, cnt_hbm,
          di_v, vals_v, ones_v, zbuf_v, agg_sh, cnt_sh):
        c = lax.axis_index("c")
        s = lax.axis_index("s")
        wid = s * NC + c

        def zrow(r, c2):
            zbuf_v[r, :] = jnp.zeros((de,), jnp.float32)
            return c2

        lax.fori_loop(0, rows_per_tile, zrow, 0)

        def orow(r, c2):
            ones_v[r, :] = jnp.full((de,), 1.0, jnp.float32)
            return c2

        lax.fori_loop(0, chunk, orow, 0)

        tile_base = s * rows_per_tile
        pltpu.sync_copy(zbuf_v, agg_sh.at[pl.ds(tile_base, rows_per_tile)])
        pltpu.sync_copy(zbuf_v, cnt_sh.at[pl.ds(tile_base, rows_per_tile)])
        plsc.subcore_barrier()

        w_base = wid * per_w

        def body(i, carry):
            base = w_base + i * chunk
            pltpu.sync_copy(dst_hbm.at[pl.ds(base, chunk)], di_v)
            pltpu.sync_copy(ea_hbm.at[pl.ds(base, chunk)], vals_v)
            pltpu.sync_copy(vals_v, agg_sh.at[di_v], add=True)
            pltpu.sync_copy(ones_v, cnt_sh.at[di_v], add=True)
            return carry

        lax.fori_loop(0, iters, body, 0)
        plsc.subcore_barrier()

        out_base = c * n_pad + tile_base
        pltpu.sync_copy(agg_sh.at[pl.ds(tile_base, rows_per_tile)],
                        agg_hbm.at[pl.ds(out_base, rows_per_tile)])
        pltpu.sync_copy(cnt_sh.at[pl.ds(tile_base, rows_per_tile)],
                        cnt_hbm.at[pl.ds(out_base, rows_per_tile)])

    return k(new_ea, dst)


def _node_mlp(pxn, aggp, cntp, w1a, b1n, w2n, b2n, block_n=2000):
    n, d = pxn.shape
    de = aggp.shape[2]

    def body(pxn_ref, aggp_ref, cntp_ref, w1a_ref, b1_ref, w2_ref, b2_ref,
             o_ref):
        agg = aggp_ref[0] + aggp_ref[1]
        cnt = cntp_ref[0, :, 0:1] + cntp_ref[1, :, 0:1]
        aggm = agg / jnp.maximum(cnt, 1.0)
        h = (pxn_ref[...]
             + jnp.dot(aggm, w1a_ref[...], preferred_element_type=jnp.float32)
             + b1_ref[...])
        h = jnp.maximum(h, 0.0)
        o_ref[...] = jnp.dot(h, w2_ref[...],
                             preferred_element_type=jnp.float32) + b2_ref[...]

    return pl.pallas_call(
        body,
        grid=(n // block_n,),
        in_specs=[
            pl.BlockSpec((block_n, d), lambda i: (i, 0)),
            pl.BlockSpec((2, block_n, de), lambda i: (0, i, 0)),
            pl.BlockSpec((2, block_n, de), lambda i: (0, i, 0)),
            pl.BlockSpec((de, d), lambda i: (0, 0)),
            pl.BlockSpec((1, d), lambda i: (0, 0)),
            pl.BlockSpec((d, d), lambda i: (0, 0)),
            pl.BlockSpec((1, d), lambda i: (0, 0)),
        ],
        out_specs=pl.BlockSpec((block_n, d), lambda i: (i, 0)),
        out_shape=jax.ShapeDtypeStruct((n, d), jnp.float32),
    )(pxn, aggp, cntp, w1a, b1n, w2n, b2n)


def kernel(x, edge_index, edge_attr, W1e, b1e, W2e, b2e, W1n, b1n, W2n, b2n):
    n, d = x.shape
    e, de = edge_attr.shape
    src = edge_index[0]
    dst = edge_index[1]

    wcat = jnp.concatenate([W1e[:d], W1e[d:2 * d], W1n[:d]], axis=1)
    ps, pd_, pxn = _proj(x, wcat)

    gsum = _gather_sum(ps, pd_, src, dst)
    new_ea, new_ea_packed = _edge_mlp(gsum, edge_attr, W1e[2 * d:],
                                      b1e.reshape(1, d), W2e,
                                      b2e.reshape(1, de))
    aggp, cntp = _scatter_partials(new_ea_packed.reshape(e, de), dst, n)
    n_pad = aggp.shape[0] // NC
    new_x = _node_mlp(pxn, aggp.reshape(NC, n_pad, de),
                      cntp.reshape(NC, n_pad, de),
                      W1n[d:], b1n.reshape(1, d), W2n, b2n.reshape(1, d))
    return (new_x, new_ea)


# speedup vs baseline: 1.2905x; 1.2905x over previous
import functools

import jax
import jax.numpy as jnp
from jax import lax
from jax.experimental import pallas as pl
from jax.experimental.pallas import tpu as pltpu
from jax.experimental.pallas import tpu_sc as plsc

NC = 2
NS = 16
LANES = 16
NW = NC * NS


def _proj_body(x_ref, w_ref, os_ref, od_ref, on_ref):
    d = x_ref.shape[1]
    r = jnp.dot(x_ref[...], w_ref[...], preferred_element_type=jnp.float32)
    os_ref[...] = r[:, :d]
    od_ref[...] = r[:, d:2 * d]
    on_ref[...] = r[:, 2 * d:]


def _proj(x, wcat, block_n=2000):
    n, d = x.shape
    k = wcat.shape[1]
    return pl.pallas_call(
        _proj_body,
        grid=(n // block_n,),
        in_specs=[
            pl.BlockSpec((block_n, d), lambda i: (i, 0)),
            pl.BlockSpec((d, k), lambda i: (0, 0)),
        ],
        out_specs=[
            pl.BlockSpec((block_n, d), lambda i: (i, 0)),
            pl.BlockSpec((block_n, d), lambda i: (i, 0)),
            pl.BlockSpec((block_n, d), lambda i: (i, 0)),
        ],
        out_shape=[
            jax.ShapeDtypeStruct((n, d), jnp.float32),
            jax.ShapeDtypeStruct((n, d), jnp.float32),
            jax.ShapeDtypeStruct((n, d), jnp.float32),
        ],
    )(x, wcat)


def _gather_sum(ps, pd, src, dst, chunk=200):
    e = src.shape[0]
    d = ps.shape[1]
    per_w = e // NW
    iters = per_w // chunk
    mesh = plsc.VectorSubcoreMesh(core_axis_name="c", subcore_axis_name="s")

    @functools.partial(
        pl.kernel, mesh=mesh,
        out_type=jax.ShapeDtypeStruct((e, d), jnp.float32),
        scratch_types=[
            [pltpu.VMEM((chunk,), jnp.int32)] * 2,
            [pltpu.VMEM((chunk,), jnp.int32)] * 2,
            [pltpu.VMEM((chunk, d), jnp.float32)] * 2,
            [pltpu.VMEM((chunk, d), jnp.float32)] * 2,
            [pltpu.SemaphoreType.DMA] * 2,
            [pltpu.SemaphoreType.DMA] * 2,
        ],
    )
    def k(ps_hbm, pd_hbm, src_hbm, dst_hbm, out_hbm,
          si_v, di_v, rs_v, rd_v, sem_s, sem_d):
        wid = lax.axis_index("s") * NC + lax.axis_index("c")
        w_base = wid * per_w

        def fire(i, b):
            base = w_base + i * chunk
            pltpu.sync_copy(src_hbm.at[pl.ds(base, chunk)], si_v[b])
            pltpu.sync_copy(dst_hbm.at[pl.ds(base, chunk)], di_v[b])
            pltpu.async_copy(ps_hbm.at[si_v[b]], rs_v[b], sem_s[b])
            pltpu.async_copy(pd_hbm.at[di_v[b]], rd_v[b], sem_d[b])

        fire(0, 0)

        @pl.loop(0, iters, step=2)
        def outer(i0):
            for b in range(2):
                i = i0 + b

                @pl.when(i < iters)
                def _():
                    @pl.when(i + 1 < iters)
                    def _():
                        fire(i + 1, 1 - b)

                    pltpu.make_async_copy(ps_hbm.at[si_v[b]], rs_v[b],
                                          sem_s[b]).wait()
                    pltpu.make_async_copy(pd_hbm.at[di_v[b]], rd_v[b],
                                          sem_d[b]).wait()

                    def addrow(r, c2):
                        for j in range(d // LANES):
                            sl = pl.ds(j * LANES, LANES)
                            rs_v[b][r, sl] = rs_v[b][r, sl] + rd_v[b][r, sl]
                        return c2

                    lax.fori_loop(0, chunk, addrow, 0)
                    base = w_base + i * chunk
                    pltpu.sync_copy(rs_v[b], out_hbm.at[pl.ds(base, chunk)])

    return k(ps, pd, src, dst)


def _edge_mlp(gsum, ea, w1a, b1e, w2e, b2e, block_e=8000):
    e, d = gsum.shape
    de = ea.shape[1]

    def body(g_ref, ea_ref, w1a_ref, b1_ref, w2_ref, b2_ref, o_ref, op_ref):
        h = (g_ref[...]
             + jnp.dot(ea_ref[...], w1a_ref[...],
                       preferred_element_type=jnp.float32)
             + b1_ref[...])
        h = jnp.maximum(h, 0.0)
        out = jnp.dot(h, w2_ref[...],
                      preferred_element_type=jnp.float32) + b2_ref[...]
        o_ref[...] = out
        out3 = out.reshape(block_e // 8, 8, de)
        op_ref[...] = jnp.concatenate([out3[:, a, :] for a in range(8)],
                                      axis=-1)

    return pl.pallas_call(
        body,
        grid=(e // block_e,),
        in_specs=[
            pl.BlockSpec((block_e, d), lambda i: (i, 0)),
            pl.BlockSpec((block_e, de), lambda i: (i, 0)),
            pl.BlockSpec((de, d), lambda i: (0, 0)),
            pl.BlockSpec((1, d), lambda i: (0, 0)),
            pl.BlockSpec((d, de), lambda i: (0, 0)),
            pl.BlockSpec((1, de), lambda i: (0, 0)),
        ],
        out_specs=[
            pl.BlockSpec((block_e, de), lambda i: (i, 0)),
            pl.BlockSpec((block_e // 8, 8 * de), lambda i: (i, 0)),
        ],
        out_shape=[
            jax.ShapeDtypeStruct((e, de), jnp.float32),
            jax.ShapeDtypeStruct((e // 8, 8 * de), jnp.float32),
        ],
    )(gsum, ea, w1a, b1e, w2e, b2e)


def _scatter_partials(new_ea, dst, n_nodes, chunk=400):
    e, de = new_ea.shape
    per_w = e // NW
    iters = per_w // chunk
    n_pad = ((n_nodes + 8 * NS - 1) // (8 * NS)) * (8 * NS)
    rows_per_tile = n_pad // NS
    mesh = plsc.VectorSubcoreMesh(core_axis_name="c", subcore_axis_name="s")

    @functools.partial(
        pl.kernel, mesh=mesh,
        out_type=[jax.ShapeDtypeStruct((NC * n_pad, de), jnp.float32),
                  jax.ShapeDtypeStruct((NC * n_pad, de), jnp.float32)],
        scratch_types=[
            pltpu.VMEM((chunk,), jnp.int32),
            pltpu.VMEM((chunk, de), jnp.float32),
            pltpu.VMEM((chunk, de), jnp.float32),
            pltpu.VMEM((rows_per_tile, de), jnp.float32),
            pltpu.VMEM_SHARED((n_pad, de), jnp.float32),
            pltpu.VMEM_SHARED((n_pad, de), jnp.float32),
        ],
        compiler_params=pltpu.CompilerParams(use_tc_tiling_on_sc=False),
    )
    def k(ea_hbm, dst_hbm, agg_hbm, cnt_hbm,
          di_v, vals_v, ones_v, zbuf_v, agg_sh, cnt_sh):
        c = lax.axis_index("c")
        s = lax.axis_index("s")
        wid = s * NC + c

        def zrow(r, c2):
            zbuf_v[r, :] = jnp.zeros((de,), jnp.float32)
            return c2

        lax.fori_loop(0, rows_per_tile, zrow, 0)

        def orow(r, c2):
            ones_v[r, :] = jnp.full((de,), 1.0, jnp.float32)
            return c2

        lax.fori_loop(0, chunk, orow, 0)

        tile_base = s * rows_per_tile
        pltpu.sync_copy(zbuf_v, agg_sh.at[pl.ds(tile_base, rows_per_tile)])
        pltpu.sync_copy(zbuf_v, cnt_sh.at[pl.ds(tile_base, rows_per_tile)])
        plsc.subcore_barrier()

        w_base = wid * per_w

        def body(i, carry):
            base = w_base + i * chunk
            pltpu.sync_copy(dst_hbm.at[pl.ds(base, chunk)], di_v)
            pltpu.sync_copy(ea_hbm.at[pl.ds(base, chunk)], vals_v)
            pltpu.sync_copy(vals_v, agg_sh.at[di_v], add=True)
            pltpu.sync_copy(ones_v, cnt_sh.at[di_v], add=True)
            return carry

        lax.fori_loop(0, iters, body, 0)
        plsc.subcore_barrier()

        out_base = c * n_pad + tile_base
        pltpu.sync_copy(agg_sh.at[pl.ds(tile_base, rows_per_tile)],
                        agg_hbm.at[pl.ds(out_base, rows_per_tile)])
        pltpu.sync_copy(cnt_sh.at[pl.ds(tile_base, rows_per_tile)],
                        cnt_hbm.at[pl.ds(out_base, rows_per_tile)])

    return k(new_ea, dst)


def _node_mlp(pxn, aggp, cntp, w1a, b1n, w2n, b2n, block_n=2000):
    n, d = pxn.shape
    de = aggp.shape[2]

    def body(pxn_ref, aggp_ref, cntp_ref, w1a_ref, b1_ref, w2_ref, b2_ref,
             o_ref):
        agg = aggp_ref[0] + aggp_ref[1]
        cnt = cntp_ref[0, :, 0:1] + cntp_ref[1, :, 0:1]
        aggm = agg / jnp.maximum(cnt, 1.0)
        h = (pxn_ref[...]
             + jnp.dot(aggm, w1a_ref[...], preferred_element_type=jnp.float32)
             + b1_ref[...])
        h = jnp.maximum(h, 0.0)
        o_ref[...] = jnp.dot(h, w2_ref[...],
                             preferred_element_type=jnp.float32) + b2_ref[...]

    return pl.pallas_call(
        body,
        grid=(n // block_n,),
        in_specs=[
            pl.BlockSpec((block_n, d), lambda i: (i, 0)),
            pl.BlockSpec((2, block_n, de), lambda i: (0, i, 0)),
            pl.BlockSpec((2, block_n, de), lambda i: (0, i, 0)),
            pl.BlockSpec((de, d), lambda i: (0, 0)),
            pl.BlockSpec((1, d), lambda i: (0, 0)),
            pl.BlockSpec((d, d), lambda i: (0, 0)),
            pl.BlockSpec((1, d), lambda i: (0, 0)),
        ],
        out_specs=pl.BlockSpec((block_n, d), lambda i: (i, 0)),
        out_shape=jax.ShapeDtypeStruct((n, d), jnp.float32),
    )(pxn, aggp, cntp, w1a, b1n, w2n, b2n)


def kernel(x, edge_index, edge_attr, W1e, b1e, W2e, b2e, W1n, b1n, W2n, b2n):
    n, d = x.shape
    e, de = edge_attr.shape
    src = edge_index[0]
    dst = edge_index[1]

    wcat = jnp.concatenate([W1e[:d], W1e[d:2 * d], W1n[:d]], axis=1)
    ps, pd_, pxn = _proj(x, wcat)

    gsum = _gather_sum(ps, pd_, src, dst)
    new_ea, new_ea_packed = _edge_mlp(gsum, edge_attr, W1e[2 * d:],
                                      b1e.reshape(1, d), W2e,
                                      b2e.reshape(1, de))
    aggp, cntp = _scatter_partials(new_ea_packed.reshape(e, de), dst, n)
    n_pad = aggp.shape[0] // NC
    new_x = _node_mlp(pxn, aggp.reshape(NC, n_pad, de),
                      cntp.reshape(NC, n_pad, de),
                      W1n[d:], b1n.reshape(1, d), W2n, b2n.reshape(1, d))
    return (new_x, new_ea)
